# baseline (speedup 1.0000x reference)
import functools

import jax
import jax.numpy as jnp
from jax import lax
from jax.experimental import pallas as pl
from jax.experimental.pallas import tpu as pltpu
from jax.experimental.pallas import tpu_sc as plsc

B, N, K, D, DH = 8, 2048, 16, 37, 200
NC, NS = 2, 16
QPB = (NC * NS) // B
ROWS = N // QPB
TW = N * D
CH = ROWS * D
GRP = ROWS // 16


def _gather_round(t_ref, a_ref, e_ref):
    lane = lax.iota(jnp.int32, 16)

    @plsc.parallel_loop(0, GRP)
    def body(g):
        node_idx = (lane + g * 16) * D
        dc = 5
        for d0 in range(0, D, dc):
            dn = min(dc, D - d0)
            accs = [None] * dn
            for k in range(K):
                off = e_ref[pl.ds(k * ROWS + g * 16, 16)] * D
                for i in range(dn):
                    val = plsc.load_gather(t_ref, [off + (d0 + i)])
                    accs[i] = val if k == 0 else accs[i] + val
            for i in range(dn):
                plsc.store_scatter(a_ref, [node_idx + (d0 + i)], accs[i])


def _sc_body(src_hbm, ed_hbm, out_hbm, t_ref, a_ref, e_ref):
    c = lax.axis_index("c")
    s = lax.axis_index("s")
    b = c * (B // NC) + s // QPB
    q = s % QPB

    pltpu.sync_copy(src_hbm.at[b], t_ref)
    pltpu.sync_copy(ed_hbm.at[b, q], e_ref)
    _gather_round(t_ref, a_ref, e_ref)
    pltpu.sync_copy(a_ref, out_hbm.at[b, pl.ds(q * CH, CH)])


_sc_gather = functools.partial(
    pl.kernel,
    out_type=jax.ShapeDtypeStruct((B, TW), jnp.float32),
    mesh=plsc.VectorSubcoreMesh(core_axis_name="c", subcore_axis_name="s"),
    compiler_params=pltpu.CompilerParams(needs_layout_passes=False),
    scratch_types=[
        pltpu.VMEM((TW,), jnp.float32),
        pltpu.VMEM((CH,), jnp.float32),
        pltpu.VMEM((K * ROWS,), jnp.int32),
    ],
)(_sc_body)


def _tc_pre_body(pa_ref, wse_ref, bse_ref, bne_ref, wsd_ref, bsd_ref,
                 s_ref, self_ref):
    pa = pa_ref[0]
    wse = wse_ref[...]
    bse = bse_ref[...]

    s_ref[0] = (jnp.dot(pa, wse, preferred_element_type=jnp.float32)
                + bse + bne_ref[...])

    wfs = jnp.dot(wse, wsd_ref[...], preferred_element_type=jnp.float32)
    bias_s = (jnp.dot(bse, wsd_ref[...], preferred_element_type=jnp.float32)
              + bsd_ref[...])
    self_ref[0] = jnp.dot(pa, wfs, preferred_element_type=jnp.float32) + bias_s


def _tc_pre(pa, wse, bse, bne, wsd, bsd):
    full = lambda shape: pl.BlockSpec(shape, lambda i: (0, 0))
    return pl.pallas_call(
        _tc_pre_body,
        grid=(B,),
        in_specs=[
            pl.BlockSpec((1, N, D), lambda i: (i, 0, 0)),
            full((D, DH)), full((1, DH)), full((1, DH)),
            full((DH, D)), full((1, D)),
        ],
        out_specs=[
            pl.BlockSpec((1, N, DH), lambda i: (i, 0, 0)),
            pl.BlockSpec((1, N, D), lambda i: (i, 0, 0)),
        ],
        out_shape=[
            jax.ShapeDtypeStruct((B, N, DH), jnp.float32),
            jax.ShapeDtypeStruct((B, N, D), jnp.float32),
        ],
    )(pa, wse, bse, bne, wsd, bsd)


def _tc_enc_body(g1_ref, s_ref, wne_ref, sum_ref):
    enc = (jnp.dot(g1_ref[0], wne_ref[...], preferred_element_type=jnp.float32)
           + s_ref[0])
    sum_ref[0] = jnp.maximum(enc, 0.0)


def _tc_enc(g1, s, wne):
    return pl.pallas_call(
        _tc_enc_body,
        grid=(B,),
        in_specs=[
            pl.BlockSpec((1, N, D), lambda i: (i, 0, 0)),
            pl.BlockSpec((1, N, DH), lambda i: (i, 0, 0)),
            pl.BlockSpec((D, DH), lambda i: (0, 0)),
        ],
        out_specs=pl.BlockSpec((1, N, DH), lambda i: (i, 0, 0)),
        out_shape=jax.ShapeDtypeStruct((B, N, DH), jnp.float32),
    )(g1, s, wne)


def _tc_dec_body(g2_ref, wne_ref, bne_ref, wnd_ref, bnd_ref, nbr_ref):
    wnd = wnd_ref[...]
    wfe = jnp.dot(wne_ref[...], wnd, preferred_element_type=jnp.float32)
    bias_n = (float(K) * jnp.dot(bne_ref[...], wnd,
                                 preferred_element_type=jnp.float32)
              + bnd_ref[...])
    nbr_ref[0] = (jnp.dot(g2_ref[0], wfe, preferred_element_type=jnp.float32)
                  + bias_n)


def _tc_dec(g2, wne, bne, wnd, bnd):
    full = lambda shape: pl.BlockSpec(shape, lambda i: (0, 0))
    return pl.pallas_call(
        _tc_dec_body,
        grid=(B,),
        in_specs=[
            pl.BlockSpec((1, N, D), lambda i: (i, 0, 0)),
            full((D, DH)), full((1, DH)), full((DH, D)), full((1, D)),
        ],
        out_specs=pl.BlockSpec((1, N, D), lambda i: (i, 0, 0)),
        out_shape=jax.ShapeDtypeStruct((B, N, D), jnp.float32),
    )(g2, wne, bne, wnd, bnd)


def kernel(p_atoms, p_edges, W_self_e, b_self_e, W_nbr_e, b_nbr_e,
           W_nbr_d, b_nbr_d, W_self_d, b_self_d):
    ed = (p_edges.astype(jnp.int32)
          .transpose(0, 2, 1)
          .reshape(B, K, QPB, ROWS)
          .transpose(0, 2, 1, 3)
          .reshape(B, QPB, K * ROWS))
    pa2 = p_atoms.reshape(B, TW)
    g1f = _sc_gather(pa2, ed)
    g2f = _sc_gather(g1f, ed)
    g1 = g1f.reshape(B, N, D)
    g2 = g2f.reshape(B, N, D)

    s_pre, out_self = _tc_pre(
        p_atoms, W_self_e, b_self_e.reshape(1, DH), b_nbr_e.reshape(1, DH),
        W_self_d, b_self_d.reshape(1, D))
    summed = _tc_enc(g1, s_pre, W_nbr_e)
    out_nbr = _tc_dec(g2, W_nbr_e, b_nbr_e.reshape(1, DH),
                      W_nbr_d, b_nbr_d.reshape(1, D))
    return (summed, p_atoms, out_nbr, out_self)


# speedup vs baseline: 1.0272x; 1.0272x over previous
import functools

import jax
import jax.numpy as jnp
from jax import lax
from jax.experimental import pallas as pl
from jax.experimental.pallas import tpu as pltpu
from jax.experimental.pallas import tpu_sc as plsc

B, N, K, D, DH = 8, 2048, 16, 37, 200
NC, NS = 2, 16
QPB = (NC * NS) // B
ROWS = N // QPB
TW = N * D
CH = ROWS * D
GRP = ROWS // 16


def _gather_round(t_ref, a_ref, e_ref):
    lane = lax.iota(jnp.int32, 16)

    @plsc.parallel_loop(0, GRP)
    def body(g):
        node_idx = (lane + g * 16) * D
        dc = 5
        for d0 in range(0, D, dc):
            dn = min(dc, D - d0)
            accs = [None] * dn
            for k in range(K):
                off = e_ref[pl.ds(k * ROWS + g * 16, 16)] * D
                for i in range(dn):
                    val = plsc.load_gather(t_ref, [off + (d0 + i)])
                    accs[i] = val if k == 0 else accs[i] + val
            for i in range(dn):
                plsc.store_scatter(a_ref, [node_idx + (d0 + i)], accs[i])


def _sc_body(src_hbm, ed_hbm, out_hbm, t_ref, a_ref, e_ref):
    c = lax.axis_index("c")
    s = lax.axis_index("s")
    b = c * (B // NC) + s // QPB
    q = s % QPB

    pltpu.sync_copy(src_hbm.at[b], t_ref)
    pltpu.sync_copy(ed_hbm.at[b, q], e_ref)
    _gather_round(t_ref, a_ref, e_ref)
    pltpu.sync_copy(a_ref, out_hbm.at[b, pl.ds(q * CH, CH)])


_sc_gather = functools.partial(
    pl.kernel,
    out_type=jax.ShapeDtypeStruct((B, TW), jnp.float32),
    mesh=plsc.VectorSubcoreMesh(core_axis_name="c", subcore_axis_name="s"),
    compiler_params=pltpu.CompilerParams(needs_layout_passes=False),
    scratch_types=[
        pltpu.VMEM((TW,), jnp.float32),
        pltpu.VMEM((CH,), jnp.float32),
        pltpu.VMEM((K * ROWS,), jnp.int32),
    ],
)(_sc_body)


def _tc_pre_body(pa_ref, wse_ref, bse_ref, wsd_ref, bsd_ref, self_ref):
    wfs = jnp.dot(wse_ref[...], wsd_ref[...],
                  preferred_element_type=jnp.float32)
    bias_s = (jnp.dot(bse_ref[...], wsd_ref[...],
                      preferred_element_type=jnp.float32)
              + bsd_ref[...])
    self_ref[0] = (jnp.dot(pa_ref[0], wfs, preferred_element_type=jnp.float32)
                   + bias_s)


def _tc_pre(pa, wse, bse, wsd, bsd):
    full = lambda shape: pl.BlockSpec(shape, lambda i: (0, 0))
    return pl.pallas_call(
        _tc_pre_body,
        grid=(B,),
        in_specs=[
            pl.BlockSpec((1, N, D), lambda i: (i, 0, 0)),
            full((D, DH)), full((1, DH)), full((DH, D)), full((1, D)),
        ],
        out_specs=pl.BlockSpec((1, N, D), lambda i: (i, 0, 0)),
        out_shape=jax.ShapeDtypeStruct((B, N, D), jnp.float32),
    )(pa, wse, bse, wsd, bsd)


def _tc_enc_body(pa_ref, g1_ref, wse_ref, bse_ref, wne_ref, bne_ref, sum_ref):
    enc = (jnp.dot(g1_ref[0], wne_ref[...], preferred_element_type=jnp.float32)
           + jnp.dot(pa_ref[0], wse_ref[...],
                     preferred_element_type=jnp.float32)
           + bne_ref[...] + bse_ref[...])
    sum_ref[0] = jnp.maximum(enc, 0.0)


def _tc_enc(pa, g1, wse, bse, wne, bne):
    row_spec = pl.BlockSpec((1, N, D), lambda i: (i, 0, 0))
    full = lambda shape: pl.BlockSpec(shape, lambda i: (0, 0))
    return pl.pallas_call(
        _tc_enc_body,
        grid=(B,),
        in_specs=[
            row_spec, row_spec,
            full((D, DH)), full((1, DH)), full((D, DH)), full((1, DH)),
        ],
        out_specs=pl.BlockSpec((1, N, DH), lambda i: (i, 0, 0)),
        out_shape=jax.ShapeDtypeStruct((B, N, DH), jnp.float32),
    )(pa, g1, wse, bse, wne, bne)


def _tc_dec_body(g2_ref, wne_ref, bne_ref, wnd_ref, bnd_ref, nbr_ref):
    wnd = wnd_ref[...]
    wfe = jnp.dot(wne_ref[...], wnd, preferred_element_type=jnp.float32)
    bias_n = (float(K) * jnp.dot(bne_ref[...], wnd,
                                 preferred_element_type=jnp.float32)
              + bnd_ref[...])
    nbr_ref[0] = (jnp.dot(g2_ref[0], wfe, preferred_element_type=jnp.float32)
                  + bias_n)


def _tc_dec(g2, wne, bne, wnd, bnd):
    full = lambda shape: pl.BlockSpec(shape, lambda i: (0, 0))
    return pl.pallas_call(
        _tc_dec_body,
        grid=(B,),
        in_specs=[
            pl.BlockSpec((1, N, D), lambda i: (i, 0, 0)),
            full((D, DH)), full((1, DH)), full((DH, D)), full((1, D)),
        ],
        out_specs=pl.BlockSpec((1, N, D), lambda i: (i, 0, 0)),
        out_shape=jax.ShapeDtypeStruct((B, N, D), jnp.float32),
    )(g2, wne, bne, wnd, bnd)


def kernel(p_atoms, p_edges, W_self_e, b_self_e, W_nbr_e, b_nbr_e,
           W_nbr_d, b_nbr_d, W_self_d, b_self_d):
    ed = (p_edges.astype(jnp.int32)
          .transpose(0, 2, 1)
          .reshape(B, K, QPB, ROWS)
          .transpose(0, 2, 1, 3)
          .reshape(B, QPB, K * ROWS))
    pa2 = p_atoms.reshape(B, TW)
    out_self = _tc_pre(p_atoms, W_self_e, b_self_e.reshape(1, DH),
                       W_self_d, b_self_d.reshape(1, D))
    g1f = _sc_gather(pa2, ed)
    g2f = _sc_gather(g1f, ed)
    g1 = g1f.reshape(B, N, D)
    g2 = g2f.reshape(B, N, D)

    summed = _tc_enc(p_atoms, g1, W_self_e, b_self_e.reshape(1, DH),
                     W_nbr_e, b_nbr_e.reshape(1, DH))
    out_nbr = _tc_dec(g2, W_nbr_e, b_nbr_e.reshape(1, DH),
                      W_nbr_d, b_nbr_d.reshape(1, D))
    return (summed, p_atoms, out_nbr, out_self)


# speedup vs baseline: 1.1014x; 1.0722x over previous
import functools

import jax
import jax.numpy as jnp
from jax import lax
from jax.experimental import pallas as pl
from jax.experimental.pallas import tpu as pltpu
from jax.experimental.pallas import tpu_sc as plsc

B, N, K, D, DH = 8, 2048, 16, 37, 200
NC, NS = 2, 16
QPB = (NC * NS) // B
ROWS = N // QPB
TW = N * D
CH = ROWS * D
GRP = ROWS // 16


def _gather_round(t_ref, a_ref, e_ref):
    lane = lax.iota(jnp.int32, 16)

    @plsc.parallel_loop(0, GRP)
    def body(g):
        node_idx = (lane + g * 16) * D
        dc = 5
        for d0 in range(0, D, dc):
            dn = min(dc, D - d0)
            accs = [None] * dn
            for k in range(K):
                off = e_ref[pl.ds(k * ROWS + g * 16, 16)] * D
                for i in range(dn):
                    val = plsc.load_gather(t_ref, [off + (d0 + i)])
                    accs[i] = val if k == 0 else accs[i] + val
            for i in range(dn):
                plsc.store_scatter(a_ref, [node_idx + (d0 + i)], accs[i])


def _sc_body(src_hbm, ed_hbm, out_hbm, t_ref, a_ref, e_ref):
    c = lax.axis_index("c")
    s = lax.axis_index("s")
    b = c * (B // NC) + s // QPB
    q = s % QPB

    pltpu.sync_copy(src_hbm.at[b], t_ref)
    pltpu.sync_copy(ed_hbm.at[b, q], e_ref)
    _gather_round(t_ref, a_ref, e_ref)
    pltpu.sync_copy(a_ref, out_hbm.at[b, pl.ds(q * CH, CH)])


_sc_gather = functools.partial(
    pl.kernel,
    out_type=jax.ShapeDtypeStruct((B, TW), jnp.float32),
    mesh=plsc.VectorSubcoreMesh(core_axis_name="c", subcore_axis_name="s"),
    compiler_params=pltpu.CompilerParams(needs_layout_passes=False),
    scratch_types=[
        pltpu.VMEM((TW,), jnp.float32),
        pltpu.VMEM((CH,), jnp.float32),
        pltpu.VMEM((K * ROWS,), jnp.int32),
    ],
)(_sc_body)


def _tc_enc_body(pa_ref, g1_ref, wse_ref, bse_ref, wne_ref, bne_ref,
                 wsd_ref, bsd_ref, sum_ref, self_ref):
    pa = pa_ref[0]
    g1 = g1_ref[0]
    wse = wse_ref[...]
    bse = bse_ref[...]

    enc = (jnp.dot(g1, wne_ref[...], preferred_element_type=jnp.float32)
           + jnp.dot(pa, wse, preferred_element_type=jnp.float32)
           + bne_ref[...] + bse)
    sum_ref[0] = jnp.maximum(enc, 0.0)

    wfs = jnp.dot(wse, wsd_ref[...], preferred_element_type=jnp.float32)
    bias_s = (jnp.dot(bse, wsd_ref[...], preferred_element_type=jnp.float32)
              + bsd_ref[...])
    self_ref[0] = jnp.dot(pa, wfs, preferred_element_type=jnp.float32) + bias_s


def _tc_enc(pa, g1, wse, bse, wne, bne, wsd, bsd):
    row_spec = pl.BlockSpec((1, N, D), lambda i: (i, 0, 0))
    full = lambda shape: pl.BlockSpec(shape, lambda i: (0, 0))
    return pl.pallas_call(
        _tc_enc_body,
        grid=(B,),
        in_specs=[
            row_spec, row_spec,
            full((D, DH)), full((1, DH)), full((D, DH)), full((1, DH)),
            full((DH, D)), full((1, D)),
        ],
        out_specs=[
            pl.BlockSpec((1, N, DH), lambda i: (i, 0, 0)),
            pl.BlockSpec((1, N, D), lambda i: (i, 0, 0)),
        ],
        out_shape=[
            jax.ShapeDtypeStruct((B, N, DH), jnp.float32),
            jax.ShapeDtypeStruct((B, N, D), jnp.float32),
        ],
    )(pa, g1, wse, bse, wne, bne, wsd, bsd)


def _tc_dec_body(g2_ref, wne_ref, bne_ref, wnd_ref, bnd_ref, nbr_ref):
    wnd = wnd_ref[...]
    wfe = jnp.dot(wne_ref[...], wnd, preferred_element_type=jnp.float32)
    bias_n = (float(K) * jnp.dot(bne_ref[...], wnd,
                                 preferred_element_type=jnp.float32)
              + bnd_ref[...])
    nbr_ref[0] = (jnp.dot(g2_ref[0], wfe, preferred_element_type=jnp.float32)
                  + bias_n)


def _tc_dec(g2, wne, bne, wnd, bnd):
    full = lambda shape: pl.BlockSpec(shape, lambda i: (0, 0))
    return pl.pallas_call(
        _tc_dec_body,
        grid=(B,),
        in_specs=[
            pl.BlockSpec((1, N, D), lambda i: (i, 0, 0)),
            full((D, DH)), full((1, DH)), full((DH, D)), full((1, D)),
        ],
        out_specs=pl.BlockSpec((1, N, D), lambda i: (i, 0, 0)),
        out_shape=jax.ShapeDtypeStruct((B, N, D), jnp.float32),
    )(g2, wne, bne, wnd, bnd)


def kernel(p_atoms, p_edges, W_self_e, b_self_e, W_nbr_e, b_nbr_e,
           W_nbr_d, b_nbr_d, W_self_d, b_self_d):
    ed = (p_edges.astype(jnp.int32)
          .transpose(0, 2, 1)
          .reshape(B, K, QPB, ROWS)
          .transpose(0, 2, 1, 3)
          .reshape(B, QPB, K * ROWS))
    pa2 = p_atoms.reshape(B, TW)
    g1f = _sc_gather(pa2, ed)
    g2f = _sc_gather(g1f, ed)
    g1 = g1f.reshape(B, N, D)
    g2 = g2f.reshape(B, N, D)

    summed, out_self = _tc_enc(
        p_atoms, g1,
        W_self_e, b_self_e.reshape(1, DH), W_nbr_e, b_nbr_e.reshape(1, DH),
        W_self_d, b_self_d.reshape(1, D))
    out_nbr = _tc_dec(g2, W_nbr_e, b_nbr_e.reshape(1, DH),
                      W_nbr_d, b_nbr_d.reshape(1, D))
    return (summed, p_atoms, out_nbr, out_self)


# speedup vs baseline: 1.1016x; 1.0002x over previous
import functools

import jax
import jax.numpy as jnp
from jax import lax
from jax.experimental import pallas as pl
from jax.experimental.pallas import tpu as pltpu
from jax.experimental.pallas import tpu_sc as plsc

B, N, K, D, DH = 8, 2048, 16, 37, 200
NC, NS = 2, 16
QPB = (NC * NS) // B
ROWS = N // QPB
TW = N * D
CH = ROWS * D
GRP = ROWS // 16


def _gather_round(t_ref, a_ref, e_ref):
    lane = lax.iota(jnp.int32, 16)

    @plsc.parallel_loop(0, GRP)
    def body(g):
        node_idx = (lane + g * 16) * D
        dc = 5
        for d0 in range(0, D, dc):
            dn = min(dc, D - d0)
            accs = [None] * dn
            for k in range(K):
                off = e_ref[pl.ds(k * ROWS + g * 16, 16)] * D
                for i in range(dn):
                    val = plsc.load_gather(t_ref, [off + (d0 + i)])
                    accs[i] = val if k == 0 else accs[i] + val
            for i in range(dn):
                plsc.store_scatter(a_ref, [node_idx + (d0 + i)], accs[i])


def _sc_body(src_hbm, ed_hbm, out_hbm, t_ref, a_ref, e_ref):
    c = lax.axis_index("c")
    s = lax.axis_index("s")
    b = c * (B // NC) + s // QPB
    q = s % QPB

    pltpu.sync_copy(src_hbm.at[b], t_ref)
    pltpu.sync_copy(ed_hbm.at[b, q], e_ref)
    _gather_round(t_ref, a_ref, e_ref)
    pltpu.sync_copy(a_ref, out_hbm.at[b, pl.ds(q * CH, CH)])


_sc_gather = functools.partial(
    pl.kernel,
    out_type=jax.ShapeDtypeStruct((B, TW), jnp.float32),
    mesh=plsc.VectorSubcoreMesh(core_axis_name="c", subcore_axis_name="s"),
    compiler_params=pltpu.CompilerParams(needs_layout_passes=False),
    scratch_types=[
        pltpu.VMEM((TW,), jnp.float32),
        pltpu.VMEM((CH,), jnp.float32),
        pltpu.VMEM((K * ROWS,), jnp.int32),
    ],
)(_sc_body)


def _tc_enc_body(pa_ref, g1_ref, wse_ref, bse_ref, wne_ref, bne_ref,
                 wsd_ref, bsd_ref, sum_ref, self_ref):
    pa = pa_ref[0]
    g1 = g1_ref[0]
    wse = wse_ref[...]
    bse = bse_ref[...]

    enc = (jnp.dot(g1, wne_ref[...], preferred_element_type=jnp.float32)
           + jnp.dot(pa, wse, preferred_element_type=jnp.float32)
           + bne_ref[...] + bse)
    sum_ref[0] = jnp.maximum(enc, 0.0)

    wfs = jnp.dot(wse, wsd_ref[...], preferred_element_type=jnp.float32)
    bias_s = (jnp.dot(bse, wsd_ref[...], preferred_element_type=jnp.float32)
              + bsd_ref[...])
    self_ref[0] = jnp.dot(pa, wfs, preferred_element_type=jnp.float32) + bias_s


def _tc_enc(pa, g1, wse, bse, wne, bne, wsd, bsd):
    row_spec = pl.BlockSpec((1, N, D), lambda i: (i, 0, 0))
    full = lambda shape: pl.BlockSpec(shape, lambda i: (0, 0))
    return pl.pallas_call(
        _tc_enc_body,
        grid=(B,),
        in_specs=[
            row_spec, row_spec,
            full((D, DH)), full((1, DH)), full((D, DH)), full((1, DH)),
            full((DH, D)), full((1, D)),
        ],
        out_specs=[
            pl.BlockSpec((1, N, DH), lambda i: (i, 0, 0)),
            pl.BlockSpec((1, N, D), lambda i: (i, 0, 0)),
        ],
        out_shape=[
            jax.ShapeDtypeStruct((B, N, DH), jnp.float32),
            jax.ShapeDtypeStruct((B, N, D), jnp.float32),
        ],
    )(pa, g1, wse, bse, wne, bne, wsd, bsd)


def _tc_dec_body(g2_ref, wne_ref, bne_ref, wnd_ref, bnd_ref, nbr_ref):
    wnd = wnd_ref[...]
    wfe = jnp.dot(wne_ref[...], wnd, preferred_element_type=jnp.float32)
    bias_n = (float(K) * jnp.dot(bne_ref[...], wnd,
                                 preferred_element_type=jnp.float32)
              + bnd_ref[...])
    nbr_ref[0] = (jnp.dot(g2_ref[0], wfe, preferred_element_type=jnp.float32)
                  + bias_n)


def _tc_dec(g2, wne, bne, wnd, bnd):
    full = lambda shape: pl.BlockSpec(shape, lambda i: (0, 0))
    return pl.pallas_call(
        _tc_dec_body,
        grid=(B,),
        in_specs=[
            pl.BlockSpec((1, N, D), lambda i: (i, 0, 0)),
            full((D, DH)), full((1, DH)), full((DH, D)), full((1, D)),
        ],
        out_specs=pl.BlockSpec((1, N, D), lambda i: (i, 0, 0)),
        out_shape=jax.ShapeDtypeStruct((B, N, D), jnp.float32),
    )(g2, wne, bne, wnd, bnd)


def kernel(p_atoms, p_edges, W_self_e, b_self_e, W_nbr_e, b_nbr_e,
           W_nbr_d, b_nbr_d, W_self_d, b_self_d):
    ed = (p_edges.astype(jnp.int32)
          .transpose(0, 2, 1)
          .reshape(B, K, QPB, ROWS)
          .transpose(0, 2, 1, 3)
          .reshape(B, QPB, K * ROWS))
    pa2 = p_atoms.reshape(B, TW)
    g1f = _sc_gather(pa2, ed)
    g2f = _sc_gather(g1f, ed)
    g1 = g1f.reshape(B, N, D)
    g2 = g2f.reshape(B, N, D)

    out_nbr = _tc_dec(g2, W_nbr_e, b_nbr_e.reshape(1, DH),
                      W_nbr_d, b_nbr_d.reshape(1, D))
    summed, out_self = _tc_enc(
        p_atoms, g1,
        W_self_e, b_self_e.reshape(1, DH), W_nbr_e, b_nbr_e.reshape(1, DH),
        W_self_d, b_self_d.reshape(1, D))
    return (summed, p_atoms, out_nbr, out_self)
